# baseline (speedup 1.0000x reference)
import functools

import jax
import jax.numpy as jnp
from jax import lax
from jax.experimental import pallas as pl
from jax.experimental.pallas import tpu as pltpu
from jax.experimental.pallas import tpu_sc as plsc

HID = 1024
FFN = 4096
NE = 8
TM = 256
NB = 24
NP = NB * TM
FC = 2048
NF = FFN // FC

NW = 32


def _router_body(x_ref, gw_ref, logits_ref, w_ref, idx_ref):
    x = x_ref[...].astype(jnp.bfloat16)
    gw = gw_ref[...].astype(jnp.bfloat16)
    logits = lax.dot_general(x, gw, (((1,), (1,)), ((), ())),
                             preferred_element_type=jnp.float32)
    logits_ref[...] = logits
    m = jnp.max(logits, axis=1, keepdims=True)
    p = jnp.exp(logits - m)
    probs = p / jnp.sum(p, axis=1, keepdims=True)
    ii = lax.broadcasted_iota(jnp.int32, probs.shape, 1)
    m1 = jnp.max(probs, axis=1, keepdims=True)
    i1 = jnp.min(jnp.where(probs == m1, ii, NE), axis=1, keepdims=True)
    probs2 = jnp.where(ii == i1, -1.0, probs)
    m2 = jnp.max(probs2, axis=1, keepdims=True)
    i2 = jnp.min(jnp.where(probs2 == m2, ii, NE), axis=1, keepdims=True)
    s = m1 + m2
    w_ref[...] = jnp.concatenate([m1 / s, m2 / s], axis=1)
    idx_ref[...] = jnp.concatenate([i1, i2], axis=1).astype(jnp.int32)


def _router(x2d, gate_w):
    t = x2d.shape[0]
    return pl.pallas_call(
        _router_body,
        out_shape=[
            jax.ShapeDtypeStruct((t, NE), jnp.float32),
            jax.ShapeDtypeStruct((t, 2), jnp.float32),
            jax.ShapeDtypeStruct((t, 2), jnp.int32),
        ],
    )(x2d, gate_w)


def _logistics(idx):
    t = idx.shape[0]
    ex = idx.reshape(-1)
    oh = (ex[:, None] == jnp.arange(NE, dtype=jnp.int32)[None, :])
    csum = jnp.cumsum(oh.astype(jnp.int32), axis=0)
    counts = csum[-1]
    rank = jnp.take_along_axis(csum, ex[:, None], axis=1)[:, 0] - 1
    nblk = (counts + TM - 1) // TM
    bstart = jnp.cumsum(nblk)
    pstart = (bstart - nblk) * TM
    dest = pstart[ex] + rank
    blk_expert = jnp.minimum(
        jnp.searchsorted(bstart, jnp.arange(NB, dtype=jnp.int32),
                         side="right").astype(jnp.int32), NE - 1)
    d_even = dest[0::2]
    d_odd = dest[1::2]
    tpw = t // NW
    return (d_even.reshape(NW, tpw), d_odd.reshape(NW, tpw),
            blk_expert, d_even, d_odd)


def _sc_dispatch(x2d, d_even, d_odd):
    t = x2d.shape[0]
    tpw = t // NW
    mesh = plsc.VectorSubcoreMesh(core_axis_name="c", subcore_axis_name="s")

    @functools.partial(
        pl.kernel,
        out_type=jax.ShapeDtypeStruct((NP, HID), jnp.float32),
        mesh=mesh,
        scratch_types=[
            pltpu.VMEM((tpw,), jnp.int32),
            pltpu.VMEM((tpw,), jnp.int32),
            pltpu.VMEM((tpw, HID), jnp.float32),
            pltpu.SemaphoreType.DMA,
            pltpu.SemaphoreType.DMA,
        ],
    )
    def k(x_hbm, de_hbm, do_hbm, out_hbm, ie_v, io_v, buf, s0, s1):
        wid = lax.axis_index("s") * 2 + lax.axis_index("c")
        pltpu.sync_copy(de_hbm.at[wid], ie_v)
        pltpu.sync_copy(do_hbm.at[wid], io_v)
        pltpu.sync_copy(x_hbm.at[pl.ds(wid * tpw, tpw)], buf)
        c0 = pltpu.async_copy(buf, out_hbm.at[ie_v], s0)
        c1 = pltpu.async_copy(buf, out_hbm.at[io_v], s1)
        c0.wait()
        c1.wait()

    return k(x2d, d_even, d_odd)


def _ffn1_body(be_ref, xg_ref, w1_ref, w3_ref, h_ref):
    xb = xg_ref[...].astype(jnp.bfloat16)
    w1 = w1_ref[0].astype(jnp.bfloat16)
    w3 = w3_ref[0].astype(jnp.bfloat16)
    a = lax.dot_general(xb, w1, (((1,), (1,)), ((), ())),
                        preferred_element_type=jnp.float32)
    b = lax.dot_general(xb, w3, (((1,), (1,)), ((), ())),
                        preferred_element_type=jnp.float32)
    h_ref[...] = ((a * lax.logistic(a)) * b).astype(jnp.bfloat16)


def _ffn1(xg, w1, w3, blk_expert):
    grid = (NF, NB)
    return pl.pallas_call(
        _ffn1_body,
        grid_spec=pltpu.PrefetchScalarGridSpec(
            num_scalar_prefetch=1,
            grid=grid,
            in_specs=[
                pl.BlockSpec((TM, HID), lambda f, j, be: (j, 0)),
                pl.BlockSpec((1, FC, HID), lambda f, j, be: (be[j], f, 0)),
                pl.BlockSpec((1, FC, HID), lambda f, j, be: (be[j], f, 0)),
            ],
            out_specs=pl.BlockSpec((TM, FC), lambda f, j, be: (j, f)),
        ),
        out_shape=jax.ShapeDtypeStruct((NP, FFN), jnp.bfloat16),
        compiler_params=pltpu.CompilerParams(
            dimension_semantics=("arbitrary", "arbitrary")),
    )(blk_expert, xg, w1, w3)


def _ffn2_body(be_ref, h_ref, w2_ref, out_ref):
    h = h_ref[...]
    w2 = w2_ref[0].astype(jnp.bfloat16)
    out_ref[...] = lax.dot_general(h, w2, (((1,), (1,)), ((), ())),
                                   preferred_element_type=jnp.float32)


def _ffn2(h, w2, blk_expert):
    return pl.pallas_call(
        _ffn2_body,
        grid_spec=pltpu.PrefetchScalarGridSpec(
            num_scalar_prefetch=1,
            grid=(NB,),
            in_specs=[
                pl.BlockSpec((TM, FFN), lambda j, be: (j, 0)),
                pl.BlockSpec((1, HID, FFN), lambda j, be: (be[j], 0, 0)),
            ],
            out_specs=pl.BlockSpec((TM, HID), lambda j, be: (j, 0)),
        ),
        out_shape=jax.ShapeDtypeStruct((NP, HID), jnp.float32),
        compiler_params=pltpu.CompilerParams(
            dimension_semantics=("arbitrary",)),
    )(blk_expert, h, w2)


def _sc_combine(pairs, inv0, inv1, w0b, w1b):
    t = inv0.shape[0]
    tpw = t // NW
    ct = 32
    mesh = plsc.VectorSubcoreMesh(core_axis_name="c", subcore_axis_name="s")

    @functools.partial(
        pl.kernel,
        out_type=jax.ShapeDtypeStruct((t, HID), jnp.float32),
        mesh=mesh,
        scratch_types=[
            pltpu.VMEM((tpw,), jnp.int32),
            pltpu.VMEM((tpw,), jnp.int32),
            pltpu.VMEM((ct, HID), jnp.float32),
            pltpu.VMEM((ct, HID), jnp.float32),
            pltpu.VMEM((tpw, 16), jnp.float32),
            pltpu.VMEM((tpw, 16), jnp.float32),
            pltpu.SemaphoreType.DMA,
            pltpu.SemaphoreType.DMA,
        ],
    )
    def k(pairs_hbm, i0_hbm, i1_hbm, w0_hbm, w1_hbm, out_hbm,
          i0_v, i1_v, r0_v, r1_v, w0_v, w1_v, s0, s1):
        wid = lax.axis_index("s") * 2 + lax.axis_index("c")
        base = wid * tpw
        pltpu.sync_copy(i0_hbm.at[pl.ds(base, tpw)], i0_v)
        pltpu.sync_copy(i1_hbm.at[pl.ds(base, tpw)], i1_v)
        pltpu.sync_copy(w0_hbm.at[pl.ds(base, tpw)], w0_v)
        pltpu.sync_copy(w1_hbm.at[pl.ds(base, tpw)], w1_v)

        def chunk(ci, _):
            c0 = pltpu.async_copy(
                pairs_hbm.at[i0_v.at[pl.ds(ci * ct, ct)]], r0_v, s0)
            c1 = pltpu.async_copy(
                pairs_hbm.at[i1_v.at[pl.ds(ci * ct, ct)]], r1_v, s1)
            c0.wait()
            c1.wait()

            def tok(i, _):
                w0 = w0_v[ci * ct + i, :]
                w1 = w1_v[ci * ct + i, :]

                def vec(v, _):
                    col = v * 16
                    r0_v[i, pl.ds(col, 16)] = (
                        w0 * r0_v[i, pl.ds(col, 16)]
                        + w1 * r1_v[i, pl.ds(col, 16)])
                    return 0

                lax.fori_loop(0, HID // 16, vec, 0)
                return 0

            lax.fori_loop(0, ct, tok, 0)
            pltpu.sync_copy(r0_v, out_hbm.at[pl.ds(base + ci * ct, ct)])
            return 0

        lax.fori_loop(0, tpw // ct, chunk, 0)

    return k(pairs, inv0, inv1, w0b, w1b)


def kernel(hidden_states, gate_w, w1, w3, w2):
    bsz, seqlen, hdim = hidden_states.shape
    x2d = hidden_states.reshape(-1, hdim)
    logits, wtop, idx = _router(x2d, gate_w)
    d_even, d_odd, blk_expert, inv0, inv1 = _logistics(idx)
    xg = _sc_dispatch(x2d, d_even, d_odd)
    h = _ffn1(xg, w1, w3, blk_expert)
    pairs = _ffn2(h, w2, blk_expert)
    w0b = jnp.broadcast_to(wtop[:, 0:1], (wtop.shape[0], 16))
    w1b = jnp.broadcast_to(wtop[:, 1:2], (wtop.shape[0], 16))
    final2d = _sc_combine(pairs, inv0, inv1, w0b, w1b)
    return (final2d.reshape(bsz, seqlen, hdim), logits)


# speedup vs baseline: 1.0496x; 1.0496x over previous
import functools

import jax
import jax.numpy as jnp
from jax import lax
from jax.experimental import pallas as pl
from jax.experimental.pallas import tpu as pltpu
from jax.experimental.pallas import tpu_sc as plsc

HID = 1024
FFN = 4096
NE = 8
TM = 256
NB = 24
NP = NB * TM
FC = 2048
NF = FFN // FC

NW = 32


def _router_body(x_ref, gw_ref, logits_ref, w_ref, idx_ref):
    x = x_ref[...].astype(jnp.bfloat16)
    gw = gw_ref[...].astype(jnp.bfloat16)
    logits = lax.dot_general(x, gw, (((1,), (1,)), ((), ())),
                             preferred_element_type=jnp.float32)
    logits_ref[...] = logits
    m = jnp.max(logits, axis=1, keepdims=True)
    p = jnp.exp(logits - m)
    probs = p / jnp.sum(p, axis=1, keepdims=True)
    ii = lax.broadcasted_iota(jnp.int32, probs.shape, 1)
    m1 = jnp.max(probs, axis=1, keepdims=True)
    i1 = jnp.min(jnp.where(probs == m1, ii, NE), axis=1, keepdims=True)
    probs2 = jnp.where(ii == i1, -1.0, probs)
    m2 = jnp.max(probs2, axis=1, keepdims=True)
    i2 = jnp.min(jnp.where(probs2 == m2, ii, NE), axis=1, keepdims=True)
    s = m1 + m2
    w_ref[...] = jnp.concatenate([m1 / s, m2 / s], axis=1)
    idx_ref[...] = jnp.concatenate([i1, i2], axis=1).astype(jnp.int32)


def _router(x2d, gate_w):
    t = x2d.shape[0]
    return pl.pallas_call(
        _router_body,
        out_shape=[
            jax.ShapeDtypeStruct((t, NE), jnp.float32),
            jax.ShapeDtypeStruct((t, 2), jnp.float32),
            jax.ShapeDtypeStruct((t, 2), jnp.int32),
        ],
    )(x2d, gate_w)


def _logistics(idx):
    t = idx.shape[0]
    ex = idx.reshape(-1)
    oh = (ex[:, None] == jnp.arange(NE, dtype=jnp.int32)[None, :])
    csum = jnp.cumsum(oh.astype(jnp.int32), axis=0)
    counts = csum[-1]
    rank = jnp.take_along_axis(csum, ex[:, None], axis=1)[:, 0] - 1
    nblk = (counts + TM - 1) // TM
    bstart = jnp.cumsum(nblk)
    pstart = (bstart - nblk) * TM
    dest = pstart[ex] + rank
    blk_expert = jnp.minimum(
        jnp.searchsorted(bstart, jnp.arange(NB, dtype=jnp.int32),
                         side="right").astype(jnp.int32), NE - 1)
    d_even = dest[0::2]
    d_odd = dest[1::2]
    nub = bstart[-1:]
    tpw = t // NW
    return (d_even.reshape(NW, tpw), d_odd.reshape(NW, tpw),
            blk_expert, nub, d_even, d_odd)


def _sc_dispatch(x2d, d_even, d_odd):
    t = x2d.shape[0]
    tpw = t // NW
    mesh = plsc.VectorSubcoreMesh(core_axis_name="c", subcore_axis_name="s")

    @functools.partial(
        pl.kernel,
        out_type=jax.ShapeDtypeStruct((NP, HID), jnp.float32),
        mesh=mesh,
        scratch_types=[
            pltpu.VMEM((tpw,), jnp.int32),
            pltpu.VMEM((tpw,), jnp.int32),
            pltpu.VMEM((tpw, HID), jnp.float32),
            pltpu.SemaphoreType.DMA,
            pltpu.SemaphoreType.DMA,
        ],
    )
    def k(x_hbm, de_hbm, do_hbm, out_hbm, ie_v, io_v, buf, s0, s1):
        wid = lax.axis_index("s") * 2 + lax.axis_index("c")
        pltpu.sync_copy(de_hbm.at[wid], ie_v)
        pltpu.sync_copy(do_hbm.at[wid], io_v)
        pltpu.sync_copy(x_hbm.at[pl.ds(wid * tpw, tpw)], buf)
        c0 = pltpu.async_copy(buf, out_hbm.at[ie_v], s0)
        c1 = pltpu.async_copy(buf, out_hbm.at[io_v], s1)
        c0.wait()
        c1.wait()

    return k(x2d, d_even, d_odd)


def _ffn1_body(be_ref, nub_ref, xg_ref, w1_ref, w3_ref, h_ref):
    @pl.when(pl.program_id(1) < nub_ref[0])
    def _():
        xb = xg_ref[...].astype(jnp.bfloat16)
        w1 = w1_ref[0].astype(jnp.bfloat16)
        w3 = w3_ref[0].astype(jnp.bfloat16)
        a = lax.dot_general(xb, w1, (((1,), (1,)), ((), ())),
                            preferred_element_type=jnp.float32)
        b = lax.dot_general(xb, w3, (((1,), (1,)), ((), ())),
                            preferred_element_type=jnp.float32)
        h_ref[...] = ((a * lax.logistic(a)) * b).astype(jnp.bfloat16)


def _ffn1(xg, w1, w3, blk_expert, nub):
    grid = (NF, NB)
    return pl.pallas_call(
        _ffn1_body,
        grid_spec=pltpu.PrefetchScalarGridSpec(
            num_scalar_prefetch=2,
            grid=grid,
            in_specs=[
                pl.BlockSpec((TM, HID), lambda f, j, be, nu: (j, 0)),
                pl.BlockSpec((1, FC, HID), lambda f, j, be, nu: (be[j], f, 0)),
                pl.BlockSpec((1, FC, HID), lambda f, j, be, nu: (be[j], f, 0)),
            ],
            out_specs=pl.BlockSpec((TM, FC), lambda f, j, be, nu: (j, f)),
        ),
        out_shape=jax.ShapeDtypeStruct((NP, FFN), jnp.bfloat16),
        compiler_params=pltpu.CompilerParams(
            dimension_semantics=("arbitrary", "arbitrary")),
    )(blk_expert, nub, xg, w1, w3)


def _ffn2_body(be_ref, nub_ref, h_ref, w2_ref, out_ref):
    @pl.when(pl.program_id(0) < nub_ref[0])
    def _():
        h = h_ref[...]
        w2 = w2_ref[0].astype(jnp.bfloat16)
        out_ref[...] = lax.dot_general(h, w2, (((1,), (1,)), ((), ())),
                                       preferred_element_type=jnp.float32)


def _ffn2(h, w2, blk_expert, nub):
    return pl.pallas_call(
        _ffn2_body,
        grid_spec=pltpu.PrefetchScalarGridSpec(
            num_scalar_prefetch=2,
            grid=(NB,),
            in_specs=[
                pl.BlockSpec((TM, FFN), lambda j, be, nu: (j, 0)),
                pl.BlockSpec((1, HID, FFN), lambda j, be, nu: (be[j], 0, 0)),
            ],
            out_specs=pl.BlockSpec((TM, HID), lambda j, be, nu: (j, 0)),
        ),
        out_shape=jax.ShapeDtypeStruct((NP, HID), jnp.float32),
        compiler_params=pltpu.CompilerParams(
            dimension_semantics=("arbitrary",)),
    )(blk_expert, nub, h, w2)


def _sc_combine(pairs, inv0, inv1, w0b, w1b):
    t = inv0.shape[0]
    tpw = t // NW
    ct = 32
    mesh = plsc.VectorSubcoreMesh(core_axis_name="c", subcore_axis_name="s")

    @functools.partial(
        pl.kernel,
        out_type=jax.ShapeDtypeStruct((t, HID), jnp.float32),
        mesh=mesh,
        scratch_types=[
            pltpu.VMEM((tpw,), jnp.int32),
            pltpu.VMEM((tpw,), jnp.int32),
            pltpu.VMEM((ct, HID), jnp.float32),
            pltpu.VMEM((ct, HID), jnp.float32),
            pltpu.VMEM((tpw, 16), jnp.float32),
            pltpu.VMEM((tpw, 16), jnp.float32),
            pltpu.SemaphoreType.DMA,
            pltpu.SemaphoreType.DMA,
        ],
    )
    def k(pairs_hbm, i0_hbm, i1_hbm, w0_hbm, w1_hbm, out_hbm,
          i0_v, i1_v, r0_v, r1_v, w0_v, w1_v, s0, s1):
        wid = lax.axis_index("s") * 2 + lax.axis_index("c")
        base = wid * tpw
        pltpu.sync_copy(i0_hbm.at[pl.ds(base, tpw)], i0_v)
        pltpu.sync_copy(i1_hbm.at[pl.ds(base, tpw)], i1_v)
        pltpu.sync_copy(w0_hbm.at[pl.ds(base, tpw)], w0_v)
        pltpu.sync_copy(w1_hbm.at[pl.ds(base, tpw)], w1_v)

        def chunk(ci, _):
            c0 = pltpu.async_copy(
                pairs_hbm.at[i0_v.at[pl.ds(ci * ct, ct)]], r0_v, s0)
            c1 = pltpu.async_copy(
                pairs_hbm.at[i1_v.at[pl.ds(ci * ct, ct)]], r1_v, s1)
            c0.wait()
            c1.wait()

            def tok(i, _):
                w0 = w0_v[ci * ct + i, :]
                w1 = w1_v[ci * ct + i, :]

                def vec(v, _):
                    col = v * 16
                    r0_v[i, pl.ds(col, 16)] = (
                        w0 * r0_v[i, pl.ds(col, 16)]
                        + w1 * r1_v[i, pl.ds(col, 16)])
                    return 0

                lax.fori_loop(0, HID // 16, vec, 0)
                return 0

            lax.fori_loop(0, ct, tok, 0)
            pltpu.sync_copy(r0_v, out_hbm.at[pl.ds(base + ci * ct, ct)])
            return 0

        lax.fori_loop(0, tpw // ct, chunk, 0)

    return k(pairs, inv0, inv1, w0b, w1b)


def kernel(hidden_states, gate_w, w1, w3, w2):
    bsz, seqlen, hdim = hidden_states.shape
    x2d = hidden_states.reshape(-1, hdim)
    logits, wtop, idx = _router(x2d, gate_w)
    d_even, d_odd, blk_expert, nub, inv0, inv1 = _logistics(idx)
    xg = _sc_dispatch(x2d, d_even, d_odd)
    h = _ffn1(xg, w1, w3, blk_expert, nub)
    pairs = _ffn2(h, w2, blk_expert, nub)
    w0b = jnp.broadcast_to(wtop[:, 0:1], (wtop.shape[0], 16))
    w1b = jnp.broadcast_to(wtop[:, 1:2], (wtop.shape[0], 16))
    final2d = _sc_combine(pairs, inv0, inv1, w0b, w1b)
    return (final2d.reshape(bsz, seqlen, hdim), logits)


# speedup vs baseline: 1.0898x; 1.0383x over previous
import functools

import jax
import jax.numpy as jnp
from jax import lax
from jax.experimental import pallas as pl
from jax.experimental.pallas import tpu as pltpu
from jax.experimental.pallas import tpu_sc as plsc

HID = 1024
FFN = 4096
NE = 8
TM = 256
NB = 24
NP = NB * TM
FC = 2048
NF = FFN // FC

NW = 32


def _router_body(x_ref, gw_ref, logits_ref, w_ref, dest_ref, blk_ref):
    x = x_ref[...].astype(jnp.bfloat16)
    gw = gw_ref[...].astype(jnp.bfloat16)
    logits = lax.dot_general(x, gw, (((1,), (1,)), ((), ())),
                             preferred_element_type=jnp.float32)
    logits_ref[...] = logits
    m = jnp.max(logits, axis=1, keepdims=True)
    p = jnp.exp(logits - m)
    probs = p / jnp.sum(p, axis=1, keepdims=True)
    ii = lax.broadcasted_iota(jnp.int32, probs.shape, 1)
    m1 = jnp.max(probs, axis=1, keepdims=True)
    i1 = jnp.min(jnp.where(probs == m1, ii, NE), axis=1, keepdims=True)
    probs2 = jnp.where(ii == i1, -1.0, probs)
    m2 = jnp.max(probs2, axis=1, keepdims=True)
    i2 = jnp.min(jnp.where(probs2 == m2, ii, NE), axis=1, keepdims=True)
    s = m1 + m2
    w_ref[...] = jnp.concatenate([m1 / s, m2 / s], axis=1)

    t = probs.shape[0]
    ch = 128
    nch = t // ch
    oh0 = ii == i1
    oh1 = ii == i2
    ohb = oh0.astype(jnp.bfloat16) + oh1.astype(jnp.bfloat16)
    rr = lax.broadcasted_iota(jnp.int32, (ch, ch), 0)
    cc = lax.broadcasted_iota(jnp.int32, (ch, ch), 1)
    tril = jnp.where(cc < rr, 1.0, 0.0).astype(jnp.bfloat16)
    cex = []
    tots = []
    for c in range(nch):
        ohc = ohb[c * ch:(c + 1) * ch]
        cex.append(lax.dot_general(tril, ohc, (((1,), (0,)), ((), ())),
                                   preferred_element_type=jnp.float32))
        tots.append(jnp.sum(ohc.astype(jnp.float32), axis=0, keepdims=True))
    totm = jnp.concatenate(tots, axis=0)
    rr2 = lax.broadcasted_iota(jnp.int32, (nch, nch), 0)
    cc2 = lax.broadcasted_iota(jnp.int32, (nch, nch), 1)
    tril2 = jnp.where(cc2 < rr2, 1.0, 0.0).astype(jnp.bfloat16)
    offs = lax.dot_general(tril2, totm.astype(jnp.bfloat16),
                           (((1,), (0,)), ((), ())),
                           preferred_element_type=jnp.float32)
    counts = jnp.sum(totm, axis=0, keepdims=True).astype(jnp.int32)
    nblk = (counts + TM - 1) // TM
    rr3 = lax.broadcasted_iota(jnp.int32, (NE, NE), 0)
    cc3 = lax.broadcasted_iota(jnp.int32, (NE, NE), 1)
    tril3 = jnp.where(rr3 < cc3, 1.0, 0.0).astype(jnp.bfloat16)
    pstartb = lax.dot_general(nblk.astype(jnp.bfloat16), tril3,
                              (((1,), (0,)), ((), ())),
                              preferred_element_type=jnp.float32)
    pstart_rows = pstartb * float(TM)
    dcols = []
    for c in range(nch):
        call = cex[c] + offs[c:c + 1] + pstart_rows
        oh0c = oh0[c * ch:(c + 1) * ch]
        oh1c = oh1[c * ch:(c + 1) * ch]
        sel0 = jnp.sum(jnp.where(oh0c, call, 0.0), axis=1, keepdims=True)
        sel1 = jnp.sum(jnp.where(oh1c, call, 0.0), axis=1, keepdims=True)
        dcols.append(jnp.concatenate([sel0, sel1], axis=1))
    dest_ref[...] = jnp.concatenate(dcols, axis=0).astype(jnp.int32)
    bstart = (pstartb + nblk.astype(jnp.float32)).astype(jnp.int32)
    bI = lax.broadcasted_iota(jnp.int32, (NB + NE, 1), 0)
    bexp = jnp.minimum(jnp.sum((bstart <= bI).astype(jnp.int32),
                               axis=1, keepdims=True), NE - 1)
    nubv = jnp.broadcast_to(bstart[:, NE - 1:NE], (NB + NE, 1))
    blk_ref[...] = jnp.where(bI < NB, bexp, nubv)


def _router(x2d, gate_w):
    t = x2d.shape[0]
    return pl.pallas_call(
        _router_body,
        out_shape=[
            jax.ShapeDtypeStruct((t, NE), jnp.float32),
            jax.ShapeDtypeStruct((t, 2), jnp.float32),
            jax.ShapeDtypeStruct((t, 2), jnp.int32),
            jax.ShapeDtypeStruct((NB + NE, 1), jnp.int32),
        ],
    )(x2d, gate_w)


def _sc_dispatch(x2d, d_even, d_odd):
    t = x2d.shape[0]
    tpw = t // NW
    mesh = plsc.VectorSubcoreMesh(core_axis_name="c", subcore_axis_name="s")

    @functools.partial(
        pl.kernel,
        out_type=jax.ShapeDtypeStruct((NP, HID), jnp.float32),
        mesh=mesh,
        scratch_types=[
            pltpu.VMEM((tpw,), jnp.int32),
            pltpu.VMEM((tpw,), jnp.int32),
            pltpu.VMEM((tpw, HID), jnp.float32),
            pltpu.SemaphoreType.DMA,
            pltpu.SemaphoreType.DMA,
        ],
    )
    def k(x_hbm, de_hbm, do_hbm, out_hbm, ie_v, io_v, buf, s0, s1):
        wid = lax.axis_index("s") * 2 + lax.axis_index("c")
        pltpu.sync_copy(de_hbm.at[wid], ie_v)
        pltpu.sync_copy(do_hbm.at[wid], io_v)
        pltpu.sync_copy(x_hbm.at[pl.ds(wid * tpw, tpw)], buf)
        c0 = pltpu.async_copy(buf, out_hbm.at[ie_v], s0)
        c1 = pltpu.async_copy(buf, out_hbm.at[io_v], s1)
        c0.wait()
        c1.wait()

    return k(x2d, d_even, d_odd)


def _ffn1_body(be_ref, nub_ref, xg_ref, w1_ref, w3_ref, h_ref):
    @pl.when(pl.program_id(1) < nub_ref[0])
    def _():
        xb = xg_ref[...].astype(jnp.bfloat16)
        w1 = w1_ref[0].astype(jnp.bfloat16)
        w3 = w3_ref[0].astype(jnp.bfloat16)
        a = lax.dot_general(xb, w1, (((1,), (1,)), ((), ())),
                            preferred_element_type=jnp.float32)
        b = lax.dot_general(xb, w3, (((1,), (1,)), ((), ())),
                            preferred_element_type=jnp.float32)
        h_ref[...] = ((a * lax.logistic(a)) * b).astype(jnp.bfloat16)


def _ffn1(xg, w1, w3, blk_expert, nub):
    grid = (NF, NB)
    return pl.pallas_call(
        _ffn1_body,
        grid_spec=pltpu.PrefetchScalarGridSpec(
            num_scalar_prefetch=2,
            grid=grid,
            in_specs=[
                pl.BlockSpec((TM, HID), lambda f, j, be, nu: (j, 0)),
                pl.BlockSpec((1, FC, HID), lambda f, j, be, nu: (be[j], f, 0)),
                pl.BlockSpec((1, FC, HID), lambda f, j, be, nu: (be[j], f, 0)),
            ],
            out_specs=pl.BlockSpec((TM, FC), lambda f, j, be, nu: (j, f)),
        ),
        out_shape=jax.ShapeDtypeStruct((NP, FFN), jnp.bfloat16),
        compiler_params=pltpu.CompilerParams(
            dimension_semantics=("arbitrary", "arbitrary")),
    )(blk_expert, nub, xg, w1, w3)


def _ffn2_body(be_ref, nub_ref, h_ref, w2_ref, out_ref):
    @pl.when(pl.program_id(0) < nub_ref[0])
    def _():
        h = h_ref[...]
        w2 = w2_ref[0].astype(jnp.bfloat16)
        out_ref[...] = lax.dot_general(h, w2, (((1,), (1,)), ((), ())),
                                       preferred_element_type=jnp.float32)


def _ffn2(h, w2, blk_expert, nub):
    return pl.pallas_call(
        _ffn2_body,
        grid_spec=pltpu.PrefetchScalarGridSpec(
            num_scalar_prefetch=2,
            grid=(NB,),
            in_specs=[
                pl.BlockSpec((TM, FFN), lambda j, be, nu: (j, 0)),
                pl.BlockSpec((1, HID, FFN), lambda j, be, nu: (be[j], 0, 0)),
            ],
            out_specs=pl.BlockSpec((TM, HID), lambda j, be, nu: (j, 0)),
        ),
        out_shape=jax.ShapeDtypeStruct((NP, HID), jnp.float32),
        compiler_params=pltpu.CompilerParams(
            dimension_semantics=("arbitrary",)),
    )(blk_expert, nub, h, w2)


def _sc_combine(pairs, inv0, inv1, w0b, w1b):
    t = inv0.shape[0]
    tpw = t // NW
    ct = 32
    mesh = plsc.VectorSubcoreMesh(core_axis_name="c", subcore_axis_name="s")

    @functools.partial(
        pl.kernel,
        out_type=jax.ShapeDtypeStruct((t, HID), jnp.float32),
        mesh=mesh,
        scratch_types=[
            pltpu.VMEM((tpw,), jnp.int32),
            pltpu.VMEM((tpw,), jnp.int32),
            pltpu.VMEM((ct, HID), jnp.float32),
            pltpu.VMEM((ct, HID), jnp.float32),
            pltpu.VMEM((tpw, 16), jnp.float32),
            pltpu.VMEM((tpw, 16), jnp.float32),
            pltpu.SemaphoreType.DMA,
            pltpu.SemaphoreType.DMA,
        ],
    )
    def k(pairs_hbm, i0_hbm, i1_hbm, w0_hbm, w1_hbm, out_hbm,
          i0_v, i1_v, r0_v, r1_v, w0_v, w1_v, s0, s1):
        wid = lax.axis_index("s") * 2 + lax.axis_index("c")
        base = wid * tpw
        pltpu.sync_copy(i0_hbm.at[pl.ds(base, tpw)], i0_v)
        pltpu.sync_copy(i1_hbm.at[pl.ds(base, tpw)], i1_v)
        pltpu.sync_copy(w0_hbm.at[pl.ds(base, tpw)], w0_v)
        pltpu.sync_copy(w1_hbm.at[pl.ds(base, tpw)], w1_v)

        def chunk(ci, _):
            c0 = pltpu.async_copy(
                pairs_hbm.at[i0_v.at[pl.ds(ci * ct, ct)]], r0_v, s0)
            c1 = pltpu.async_copy(
                pairs_hbm.at[i1_v.at[pl.ds(ci * ct, ct)]], r1_v, s1)
            c0.wait()
            c1.wait()

            def tok(i, _):
                w0 = w0_v[ci * ct + i, :]
                w1 = w1_v[ci * ct + i, :]

                def vec(v, _):
                    col = v * 16
                    r0_v[i, pl.ds(col, 16)] = (
                        w0 * r0_v[i, pl.ds(col, 16)]
                        + w1 * r1_v[i, pl.ds(col, 16)])
                    return 0

                lax.fori_loop(0, HID // 16, vec, 0)
                return 0

            lax.fori_loop(0, ct, tok, 0)
            pltpu.sync_copy(r0_v, out_hbm.at[pl.ds(base + ci * ct, ct)])
            return 0

        lax.fori_loop(0, tpw // ct, chunk, 0)

    return k(pairs, inv0, inv1, w0b, w1b)


def kernel(hidden_states, gate_w, w1, w3, w2):
    bsz, seqlen, hdim = hidden_states.shape
    x2d = hidden_states.reshape(-1, hdim)
    logits, wtop, dest, blkarr = _router(x2d, gate_w)
    inv0 = dest[:, 0]
    inv1 = dest[:, 1]
    tpw = x2d.shape[0] // NW
    d_even = inv0.reshape(NW, tpw)
    d_odd = inv1.reshape(NW, tpw)
    blk_expert = blkarr[:NB, 0]
    nub = blkarr[NB:NB + 1, 0]
    xg = _sc_dispatch(x2d, d_even, d_odd)
    h = _ffn1(xg, w1, w3, blk_expert, nub)
    pairs = _ffn2(h, w2, blk_expert, nub)
    w0b = jnp.broadcast_to(wtop[:, 0:1], (wtop.shape[0], 16))
    w1b = jnp.broadcast_to(wtop[:, 1:2], (wtop.shape[0], 16))
    final2d = _sc_combine(pairs, inv0, inv1, w0b, w1b)
    return (final2d.reshape(bsz, seqlen, hdim), logits)


# speedup vs baseline: 1.1013x; 1.0105x over previous
import functools

import jax
import jax.numpy as jnp
from jax import lax
from jax.experimental import pallas as pl
from jax.experimental.pallas import tpu as pltpu
from jax.experimental.pallas import tpu_sc as plsc

HID = 1024
FFN = 4096
NE = 8
TM = 256
NB = 24
NP = NB * TM
FC = 2048
NF = FFN // FC

NW = 32


def _router_body(x_ref, gw_ref, logits_ref, w_ref, dest_ref, blk_ref):
    x = x_ref[...].astype(jnp.bfloat16)
    gw = gw_ref[...].astype(jnp.bfloat16)
    logits = lax.dot_general(x, gw, (((1,), (1,)), ((), ())),
                             preferred_element_type=jnp.float32)
    logits_ref[...] = logits
    m = jnp.max(logits, axis=1, keepdims=True)
    p = jnp.exp(logits - m)
    probs = p / jnp.sum(p, axis=1, keepdims=True)
    ii = lax.broadcasted_iota(jnp.int32, probs.shape, 1)
    m1 = jnp.max(probs, axis=1, keepdims=True)
    i1 = jnp.min(jnp.where(probs == m1, ii, NE), axis=1, keepdims=True)
    probs2 = jnp.where(ii == i1, -1.0, probs)
    m2 = jnp.max(probs2, axis=1, keepdims=True)
    i2 = jnp.min(jnp.where(probs2 == m2, ii, NE), axis=1, keepdims=True)
    s = m1 + m2
    w_ref[...] = jnp.concatenate([m1 / s, m2 / s], axis=1)

    t = probs.shape[0]
    ch = 128
    nch = t // ch
    oh0 = ii == i1
    oh1 = ii == i2
    ohb = oh0.astype(jnp.bfloat16) + oh1.astype(jnp.bfloat16)
    rr = lax.broadcasted_iota(jnp.int32, (ch, ch), 0)
    cc = lax.broadcasted_iota(jnp.int32, (ch, ch), 1)
    tril = jnp.where(cc < rr, 1.0, 0.0).astype(jnp.bfloat16)
    cex = []
    tots = []
    for c in range(nch):
        ohc = ohb[c * ch:(c + 1) * ch]
        cex.append(lax.dot_general(tril, ohc, (((1,), (0,)), ((), ())),
                                   preferred_element_type=jnp.float32))
        tots.append(jnp.sum(ohc.astype(jnp.float32), axis=0, keepdims=True))
    totm = jnp.concatenate(tots, axis=0)
    rr2 = lax.broadcasted_iota(jnp.int32, (nch, nch), 0)
    cc2 = lax.broadcasted_iota(jnp.int32, (nch, nch), 1)
    tril2 = jnp.where(cc2 < rr2, 1.0, 0.0).astype(jnp.bfloat16)
    offs = lax.dot_general(tril2, totm.astype(jnp.bfloat16),
                           (((1,), (0,)), ((), ())),
                           preferred_element_type=jnp.float32)
    counts = jnp.sum(totm, axis=0, keepdims=True).astype(jnp.int32)
    nblk = (counts + TM - 1) // TM
    rr3 = lax.broadcasted_iota(jnp.int32, (NE, NE), 0)
    cc3 = lax.broadcasted_iota(jnp.int32, (NE, NE), 1)
    tril3 = jnp.where(rr3 < cc3, 1.0, 0.0).astype(jnp.bfloat16)
    pstartb = lax.dot_general(nblk.astype(jnp.bfloat16), tril3,
                              (((1,), (0,)), ((), ())),
                              preferred_element_type=jnp.float32)
    pstart_rows = pstartb * float(TM)
    dcols = []
    for c in range(nch):
        call = cex[c] + offs[c:c + 1] + pstart_rows
        oh0c = oh0[c * ch:(c + 1) * ch]
        oh1c = oh1[c * ch:(c + 1) * ch]
        sel0 = jnp.sum(jnp.where(oh0c, call, 0.0), axis=1, keepdims=True)
        sel1 = jnp.sum(jnp.where(oh1c, call, 0.0), axis=1, keepdims=True)
        dcols.append(jnp.concatenate([sel0, sel1], axis=1))
    dest_ref[...] = jnp.concatenate(dcols, axis=0).astype(jnp.int32)
    bstart = (pstartb + nblk.astype(jnp.float32)).astype(jnp.int32)
    bI = lax.broadcasted_iota(jnp.int32, (NB + NE, 1), 0)
    bexp = jnp.minimum(jnp.sum((bstart <= bI).astype(jnp.int32),
                               axis=1, keepdims=True), NE - 1)
    nubv = jnp.broadcast_to(bstart[:, NE - 1:NE], (NB + NE, 1))
    blk_ref[...] = jnp.where(bI < NB, bexp, nubv)


def _router(x2d, gate_w):
    t = x2d.shape[0]
    return pl.pallas_call(
        _router_body,
        out_shape=[
            jax.ShapeDtypeStruct((t, NE), jnp.float32),
            jax.ShapeDtypeStruct((t, 2), jnp.float32),
            jax.ShapeDtypeStruct((t, 2), jnp.int32),
            jax.ShapeDtypeStruct((NB + NE, 1), jnp.int32),
        ],
    )(x2d, gate_w)


def _sc_dispatch(x2d, d_even, d_odd):
    t = x2d.shape[0]
    tpw = t // NW
    mesh = plsc.VectorSubcoreMesh(core_axis_name="c", subcore_axis_name="s")

    @functools.partial(
        pl.kernel,
        out_type=jax.ShapeDtypeStruct((NP, HID), jnp.float32),
        mesh=mesh,
        scratch_types=[
            pltpu.VMEM((tpw,), jnp.int32),
            pltpu.VMEM((tpw,), jnp.int32),
            pltpu.VMEM((tpw, HID), jnp.float32),
            pltpu.SemaphoreType.DMA,
            pltpu.SemaphoreType.DMA,
        ],
    )
    def k(x_hbm, de_hbm, do_hbm, out_hbm, ie_v, io_v, buf, s0, s1):
        wid = lax.axis_index("s") * 2 + lax.axis_index("c")
        pltpu.sync_copy(de_hbm.at[wid], ie_v)
        pltpu.sync_copy(do_hbm.at[wid], io_v)
        pltpu.sync_copy(x_hbm.at[pl.ds(wid * tpw, tpw)], buf)
        c0 = pltpu.async_copy(buf, out_hbm.at[ie_v], s0)
        c1 = pltpu.async_copy(buf, out_hbm.at[io_v], s1)
        c0.wait()
        c1.wait()

    return k(x2d, d_even, d_odd)


def _ffn1_body(be_ref, nub_ref, xg_ref, w1_ref, w3_ref, h_ref):
    @pl.when(pl.program_id(1) < nub_ref[0])
    def _():
        xb = xg_ref[...].astype(jnp.bfloat16)
        w1 = w1_ref[0].astype(jnp.bfloat16)
        w3 = w3_ref[0].astype(jnp.bfloat16)
        a = lax.dot_general(xb, w1, (((1,), (1,)), ((), ())),
                            preferred_element_type=jnp.float32)
        b = lax.dot_general(xb, w3, (((1,), (1,)), ((), ())),
                            preferred_element_type=jnp.float32)
        h_ref[...] = ((a * lax.logistic(a)) * b).astype(jnp.bfloat16)


def _ffn1(xg, w1, w3, blk_expert, nub):
    grid = (NF, NB)
    return pl.pallas_call(
        _ffn1_body,
        grid_spec=pltpu.PrefetchScalarGridSpec(
            num_scalar_prefetch=2,
            grid=grid,
            in_specs=[
                pl.BlockSpec((TM, HID), lambda f, j, be, nu: (j, 0)),
                pl.BlockSpec((1, FC, HID), lambda f, j, be, nu: (be[j], f, 0)),
                pl.BlockSpec((1, FC, HID), lambda f, j, be, nu: (be[j], f, 0)),
            ],
            out_specs=pl.BlockSpec((TM, FC), lambda f, j, be, nu: (j, f)),
        ),
        out_shape=jax.ShapeDtypeStruct((NP, FFN), jnp.bfloat16),
        compiler_params=pltpu.CompilerParams(
            dimension_semantics=("arbitrary", "arbitrary")),
    )(blk_expert, nub, xg, w1, w3)


def _ffn2_body(be_ref, nub_ref, h_ref, w2_ref, out_ref):
    @pl.when(pl.program_id(0) < nub_ref[0])
    def _():
        h = h_ref[...]
        w2 = w2_ref[0].astype(jnp.bfloat16)
        out_ref[...] = lax.dot_general(h, w2, (((1,), (1,)), ((), ())),
                                       preferred_element_type=jnp.float32)


def _ffn2(h, w2, blk_expert, nub):
    return pl.pallas_call(
        _ffn2_body,
        grid_spec=pltpu.PrefetchScalarGridSpec(
            num_scalar_prefetch=2,
            grid=(NB,),
            in_specs=[
                pl.BlockSpec((TM, FFN), lambda j, be, nu: (j, 0)),
                pl.BlockSpec((1, HID, FFN), lambda j, be, nu: (be[j], 0, 0)),
            ],
            out_specs=pl.BlockSpec((TM, HID), lambda j, be, nu: (j, 0)),
        ),
        out_shape=jax.ShapeDtypeStruct((NP, HID), jnp.float32),
        compiler_params=pltpu.CompilerParams(
            dimension_semantics=("arbitrary",)),
    )(blk_expert, nub, h, w2)


def _sc_combine(pairs, inv0, inv1, w0b, w1b):
    t = inv0.shape[0]
    tpw = t // NW
    ct = 16
    nchk = tpw // ct
    mesh = plsc.VectorSubcoreMesh(core_axis_name="c", subcore_axis_name="s")

    @functools.partial(
        pl.kernel,
        out_type=jax.ShapeDtypeStruct((t, HID), jnp.float32),
        mesh=mesh,
        scratch_types=[
            pltpu.VMEM((tpw,), jnp.int32),
            pltpu.VMEM((tpw,), jnp.int32),
            [pltpu.VMEM((ct, HID), jnp.float32) for _ in range(2)],
            [pltpu.VMEM((ct, HID), jnp.float32) for _ in range(2)],
            pltpu.VMEM((tpw, 16), jnp.float32),
            pltpu.VMEM((tpw, 16), jnp.float32),
            [pltpu.SemaphoreType.DMA for _ in range(2)],
            [pltpu.SemaphoreType.DMA for _ in range(2)],
        ],
    )
    def k(pairs_hbm, i0_hbm, i1_hbm, w0_hbm, w1_hbm, out_hbm,
          i0_v, i1_v, r0s, r1s, w0_v, w1_v, sg, so):
        wid = lax.axis_index("s") * 2 + lax.axis_index("c")
        base = wid * tpw
        pltpu.sync_copy(i0_hbm.at[pl.ds(base, tpw)], i0_v)
        pltpu.sync_copy(i1_hbm.at[pl.ds(base, tpw)], i1_v)
        pltpu.sync_copy(w0_hbm.at[pl.ds(base, tpw)], w0_v)
        pltpu.sync_copy(w1_hbm.at[pl.ds(base, tpw)], w1_v)

        def fire(ci, sl):
            c0 = pltpu.async_copy(
                pairs_hbm.at[i0_v.at[pl.ds(ci * ct, ct)]], r0s[sl], sg[sl])
            c1 = pltpu.async_copy(
                pairs_hbm.at[i1_v.at[pl.ds(ci * ct, ct)]], r1s[sl], sg[sl])
            return c0, c1

        pend = fire(0, 0)
        st = [None, None]
        for ci in range(nchk):
            sl = ci % 2
            if ci + 1 < nchk:
                if st[1 - sl] is not None:
                    st[1 - sl].wait()
                nxt = fire(ci + 1, 1 - sl)
            pend[0].wait()
            pend[1].wait()
            r0_v, r1_v = r0s[sl], r1s[sl]

            def tok(i, _):
                w0 = w0_v[ci * ct + i, :]
                w1 = w1_v[ci * ct + i, :]

                def vec(v, _):
                    col = v * 16
                    r0_v[i, pl.ds(col, 16)] = (
                        w0 * r0_v[i, pl.ds(col, 16)]
                        + w1 * r1_v[i, pl.ds(col, 16)])
                    return 0

                lax.fori_loop(0, HID // 16, vec, 0, unroll=8)
                return 0

            lax.fori_loop(0, ct, tok, 0)
            st[sl] = pltpu.async_copy(
                r0_v, out_hbm.at[pl.ds(base + ci * ct, ct)], so[sl])
            if ci + 1 < nchk:
                pend = nxt
        st[0].wait()
        st[1].wait()

    return k(pairs, inv0, inv1, w0b, w1b)


def kernel(hidden_states, gate_w, w1, w3, w2):
    bsz, seqlen, hdim = hidden_states.shape
    x2d = hidden_states.reshape(-1, hdim)
    logits, wtop, dest, blkarr = _router(x2d, gate_w)
    inv0 = dest[:, 0]
    inv1 = dest[:, 1]
    tpw = x2d.shape[0] // NW
    d_even = inv0.reshape(NW, tpw)
    d_odd = inv1.reshape(NW, tpw)
    blk_expert = blkarr[:NB, 0]
    nub = blkarr[NB:NB + 1, 0]
    xg = _sc_dispatch(x2d, d_even, d_odd)
    h = _ffn1(xg, w1, w3, blk_expert, nub)
    pairs = _ffn2(h, w2, blk_expert, nub)
    w0b = jnp.broadcast_to(wtop[:, 0:1], (wtop.shape[0], 16))
    w1b = jnp.broadcast_to(wtop[:, 1:2], (wtop.shape[0], 16))
    final2d = _sc_combine(pairs, inv0, inv1, w0b, w1b)
    return (final2d.reshape(bsz, seqlen, hdim), logits)


# speedup vs baseline: 1.1097x; 1.0077x over previous
import functools

import jax
import jax.numpy as jnp
from jax import lax
from jax.experimental import pallas as pl
from jax.experimental.pallas import tpu as pltpu
from jax.experimental.pallas import tpu_sc as plsc

HID = 1024
FFN = 4096
NE = 8
TM = 256
NB = 24
NP = NB * TM
FC = 2048
NF = FFN // FC

NW = 32


def _router_body(x_ref, gw_ref, logits_ref, w0_ref, w1_ref, d0_ref, d1_ref, blk_ref):
    x = x_ref[...].astype(jnp.bfloat16)
    gw = gw_ref[...].astype(jnp.bfloat16)
    logits = lax.dot_general(x, gw, (((1,), (1,)), ((), ())),
                             preferred_element_type=jnp.float32)
    logits_ref[...] = logits
    m = jnp.max(logits, axis=1, keepdims=True)
    p = jnp.exp(logits - m)
    probs = p / jnp.sum(p, axis=1, keepdims=True)
    ii = lax.broadcasted_iota(jnp.int32, probs.shape, 1)
    m1 = jnp.max(probs, axis=1, keepdims=True)
    i1 = jnp.min(jnp.where(probs == m1, ii, NE), axis=1, keepdims=True)
    probs2 = jnp.where(ii == i1, -1.0, probs)
    m2 = jnp.max(probs2, axis=1, keepdims=True)
    i2 = jnp.min(jnp.where(probs2 == m2, ii, NE), axis=1, keepdims=True)
    s = m1 + m2
    w0_ref[...] = jnp.broadcast_to(m1 / s, (m1.shape[0], 16))
    w1_ref[...] = jnp.broadcast_to(m2 / s, (m1.shape[0], 16))

    t = probs.shape[0]
    ch = 128
    nch = t // ch
    oh0 = ii == i1
    oh1 = ii == i2
    ohb = oh0.astype(jnp.bfloat16) + oh1.astype(jnp.bfloat16)
    rr = lax.broadcasted_iota(jnp.int32, (ch, ch), 0)
    cc = lax.broadcasted_iota(jnp.int32, (ch, ch), 1)
    tril = jnp.where(cc < rr, 1.0, 0.0).astype(jnp.bfloat16)
    cex = []
    tots = []
    for c in range(nch):
        ohc = ohb[c * ch:(c + 1) * ch]
        cex.append(lax.dot_general(tril, ohc, (((1,), (0,)), ((), ())),
                                   preferred_element_type=jnp.float32))
        tots.append(jnp.sum(ohc.astype(jnp.float32), axis=0, keepdims=True))
    totm = jnp.concatenate(tots, axis=0)
    rr2 = lax.broadcasted_iota(jnp.int32, (nch, nch), 0)
    cc2 = lax.broadcasted_iota(jnp.int32, (nch, nch), 1)
    tril2 = jnp.where(cc2 < rr2, 1.0, 0.0).astype(jnp.bfloat16)
    offs = lax.dot_general(tril2, totm.astype(jnp.bfloat16),
                           (((1,), (0,)), ((), ())),
                           preferred_element_type=jnp.float32)
    counts = jnp.sum(totm, axis=0, keepdims=True).astype(jnp.int32)
    nblk = (counts + TM - 1) // TM
    rr3 = lax.broadcasted_iota(jnp.int32, (NE, NE), 0)
    cc3 = lax.broadcasted_iota(jnp.int32, (NE, NE), 1)
    tril3 = jnp.where(rr3 < cc3, 1.0, 0.0).astype(jnp.bfloat16)
    pstartb = lax.dot_general(nblk.astype(jnp.bfloat16), tril3,
                              (((1,), (0,)), ((), ())),
                              preferred_element_type=jnp.float32)
    pstart_rows = pstartb * float(TM)
    d0cols = []
    d1cols = []
    for c in range(nch):
        call = cex[c] + offs[c:c + 1] + pstart_rows
        oh0c = oh0[c * ch:(c + 1) * ch]
        oh1c = oh1[c * ch:(c + 1) * ch]
        d0cols.append(jnp.sum(jnp.where(oh0c, call, 0.0), axis=1, keepdims=True))
        d1cols.append(jnp.sum(jnp.where(oh1c, call, 0.0), axis=1, keepdims=True))
    d0_ref[...] = jnp.concatenate(d0cols, axis=0).astype(jnp.int32)
    d1_ref[...] = jnp.concatenate(d1cols, axis=0).astype(jnp.int32)
    bstart = (pstartb + nblk.astype(jnp.float32)).astype(jnp.int32)
    bI = lax.broadcasted_iota(jnp.int32, (NB + NE, 1), 0)
    bexp = jnp.minimum(jnp.sum((bstart <= bI).astype(jnp.int32),
                               axis=1, keepdims=True), NE - 1)
    nubv = jnp.broadcast_to(bstart[:, NE - 1:NE], (NB + NE, 1))
    blk_ref[...] = jnp.where(bI < NB, bexp, nubv)


def _router(x2d, gate_w):
    t = x2d.shape[0]
    return pl.pallas_call(
        _router_body,
        out_shape=[
            jax.ShapeDtypeStruct((t, NE), jnp.float32),
            jax.ShapeDtypeStruct((t, 16), jnp.float32),
            jax.ShapeDtypeStruct((t, 16), jnp.float32),
            jax.ShapeDtypeStruct((t, 1), jnp.int32),
            jax.ShapeDtypeStruct((t, 1), jnp.int32),
            jax.ShapeDtypeStruct((NB + NE, 1), jnp.int32),
        ],
    )(x2d, gate_w)


def _sc_dispatch(x2d, d_even, d_odd):
    t = x2d.shape[0]
    tpw = t // NW
    mesh = plsc.VectorSubcoreMesh(core_axis_name="c", subcore_axis_name="s")

    @functools.partial(
        pl.kernel,
        out_type=jax.ShapeDtypeStruct((NP, HID), jnp.float32),
        mesh=mesh,
        scratch_types=[
            pltpu.VMEM((tpw,), jnp.int32),
            pltpu.VMEM((tpw,), jnp.int32),
            pltpu.VMEM((tpw, HID), jnp.float32),
            pltpu.SemaphoreType.DMA,
            pltpu.SemaphoreType.DMA,
        ],
    )
    def k(x_hbm, de_hbm, do_hbm, out_hbm, ie_v, io_v, buf, s0, s1):
        wid = lax.axis_index("s") * 2 + lax.axis_index("c")
        pltpu.sync_copy(de_hbm.at[wid], ie_v)
        pltpu.sync_copy(do_hbm.at[wid], io_v)
        pltpu.sync_copy(x_hbm.at[pl.ds(wid * tpw, tpw)], buf)
        c0 = pltpu.async_copy(buf, out_hbm.at[ie_v], s0)
        c1 = pltpu.async_copy(buf, out_hbm.at[io_v], s1)
        c0.wait()
        c1.wait()

    return k(x2d, d_even, d_odd)


def _ffn1_body(be_ref, nub_ref, xg_ref, w1_ref, w3_ref, h_ref):
    @pl.when(pl.program_id(1) < nub_ref[0])
    def _():
        xb = xg_ref[...].astype(jnp.bfloat16)
        w1 = w1_ref[0].astype(jnp.bfloat16)
        w3 = w3_ref[0].astype(jnp.bfloat16)
        a = lax.dot_general(xb, w1, (((1,), (1,)), ((), ())),
                            preferred_element_type=jnp.float32)
        b = lax.dot_general(xb, w3, (((1,), (1,)), ((), ())),
                            preferred_element_type=jnp.float32)
        h_ref[...] = ((a * lax.logistic(a)) * b).astype(jnp.bfloat16)


def _ffn1(xg, w1, w3, blk_expert, nub):
    grid = (NF, NB)
    return pl.pallas_call(
        _ffn1_body,
        grid_spec=pltpu.PrefetchScalarGridSpec(
            num_scalar_prefetch=2,
            grid=grid,
            in_specs=[
                pl.BlockSpec((TM, HID), lambda f, j, be, nu: (j, 0)),
                pl.BlockSpec((1, FC, HID), lambda f, j, be, nu: (be[j], f, 0)),
                pl.BlockSpec((1, FC, HID), lambda f, j, be, nu: (be[j], f, 0)),
            ],
            out_specs=pl.BlockSpec((TM, FC), lambda f, j, be, nu: (j, f)),
        ),
        out_shape=jax.ShapeDtypeStruct((NP, FFN), jnp.bfloat16),
        compiler_params=pltpu.CompilerParams(
            dimension_semantics=("arbitrary", "arbitrary")),
    )(blk_expert, nub, xg, w1, w3)


def _ffn2_body(be_ref, nub_ref, h_ref, w2_ref, out_ref):
    @pl.when(pl.program_id(0) < nub_ref[0])
    def _():
        h = h_ref[...]
        w2 = w2_ref[0].astype(jnp.bfloat16)
        out_ref[...] = lax.dot_general(h, w2, (((1,), (1,)), ((), ())),
                                       preferred_element_type=jnp.float32)


def _ffn2(h, w2, blk_expert, nub):
    return pl.pallas_call(
        _ffn2_body,
        grid_spec=pltpu.PrefetchScalarGridSpec(
            num_scalar_prefetch=2,
            grid=(NB,),
            in_specs=[
                pl.BlockSpec((TM, FFN), lambda j, be, nu: (j, 0)),
                pl.BlockSpec((1, HID, FFN), lambda j, be, nu: (be[j], 0, 0)),
            ],
            out_specs=pl.BlockSpec((TM, HID), lambda j, be, nu: (j, 0)),
        ),
        out_shape=jax.ShapeDtypeStruct((NP, HID), jnp.float32),
        compiler_params=pltpu.CompilerParams(
            dimension_semantics=("arbitrary",)),
    )(blk_expert, nub, h, w2)


def _sc_combine(pairs, inv0, inv1, w0b, w1b):
    t = inv0.shape[0]
    tpw = t // NW
    ct = 16
    nchk = tpw // ct
    mesh = plsc.VectorSubcoreMesh(core_axis_name="c", subcore_axis_name="s")

    @functools.partial(
        pl.kernel,
        out_type=jax.ShapeDtypeStruct((t, HID), jnp.float32),
        mesh=mesh,
        scratch_types=[
            pltpu.VMEM((tpw,), jnp.int32),
            pltpu.VMEM((tpw,), jnp.int32),
            [pltpu.VMEM((ct, HID), jnp.float32) for _ in range(2)],
            [pltpu.VMEM((ct, HID), jnp.float32) for _ in range(2)],
            pltpu.VMEM((tpw, 16), jnp.float32),
            pltpu.VMEM((tpw, 16), jnp.float32),
            [pltpu.SemaphoreType.DMA for _ in range(2)],
            [pltpu.SemaphoreType.DMA for _ in range(2)],
        ],
    )
    def k(pairs_hbm, i0_hbm, i1_hbm, w0_hbm, w1_hbm, out_hbm,
          i0_v, i1_v, r0s, r1s, w0_v, w1_v, sg, so):
        wid = lax.axis_index("s") * 2 + lax.axis_index("c")
        base = wid * tpw
        pltpu.sync_copy(i0_hbm.at[pl.ds(base, tpw)], i0_v)
        pltpu.sync_copy(i1_hbm.at[pl.ds(base, tpw)], i1_v)
        pltpu.sync_copy(w0_hbm.at[pl.ds(base, tpw)], w0_v)
        pltpu.sync_copy(w1_hbm.at[pl.ds(base, tpw)], w1_v)

        def fire(ci, sl):
            c0 = pltpu.async_copy(
                pairs_hbm.at[i0_v.at[pl.ds(ci * ct, ct)]], r0s[sl], sg[sl])
            c1 = pltpu.async_copy(
                pairs_hbm.at[i1_v.at[pl.ds(ci * ct, ct)]], r1s[sl], sg[sl])
            return c0, c1

        pend = fire(0, 0)
        st = [None, None]
        for ci in range(nchk):
            sl = ci % 2
            if ci + 1 < nchk:
                if st[1 - sl] is not None:
                    st[1 - sl].wait()
                nxt = fire(ci + 1, 1 - sl)
            pend[0].wait()
            pend[1].wait()
            r0_v, r1_v = r0s[sl], r1s[sl]

            def tok(i, _):
                w0 = w0_v[ci * ct + i, :]
                w1 = w1_v[ci * ct + i, :]

                def vec(v, _):
                    col = v * 16
                    r0_v[i, pl.ds(col, 16)] = (
                        w0 * r0_v[i, pl.ds(col, 16)]
                        + w1 * r1_v[i, pl.ds(col, 16)])
                    return 0

                lax.fori_loop(0, HID // 16, vec, 0, unroll=8)
                return 0

            lax.fori_loop(0, ct, tok, 0)
            st[sl] = pltpu.async_copy(
                r0_v, out_hbm.at[pl.ds(base + ci * ct, ct)], so[sl])
            if ci + 1 < nchk:
                pend = nxt
        st[0].wait()
        st[1].wait()

    return k(pairs, inv0, inv1, w0b, w1b)


def kernel(hidden_states, gate_w, w1, w3, w2):
    bsz, seqlen, hdim = hidden_states.shape
    x2d = hidden_states.reshape(-1, hdim)
    logits, w0b, w1b, dest0, dest1, blkarr = _router(x2d, gate_w)
    t = x2d.shape[0]
    tpw = t // NW
    inv0 = dest0.reshape(t)
    inv1 = dest1.reshape(t)
    d_even = dest0.reshape(NW, tpw)
    d_odd = dest1.reshape(NW, tpw)
    blk_expert = blkarr[:NB, 0]
    nub = blkarr[NB:NB + 1, 0]
    xg = _sc_dispatch(x2d, d_even, d_odd)
    h = _ffn1(xg, w1, w3, blk_expert, nub)
    pairs = _ffn2(h, w2, blk_expert, nub)
    final2d = _sc_combine(pairs, inv0, inv1, w0b, w1b)
    return (final2d.reshape(bsz, seqlen, hdim), logits)
